# baseline (speedup 1.0000x reference)
import jax
import jax.numpy as jnp
from jax import lax
from jax.experimental import pallas as pl
from jax.experimental.pallas import tpu as pltpu

VOCAB_SIZE = 1000
DEPTH = VOCAB_SIZE + 1
MASK_TOKEN = -1
PAD_TOKEN = -2

BATCH = 1024
SEQ = 50
BBLK = 64
GRID = BATCH // BBLK


def _ohe_tc_body(x_ref, m_ref, out_ref):
    xi = x_ref[...]
    xi = jnp.where(xi == PAD_TOKEN, VOCAB_SIZE, xi)
    m = m_ref[...]
    bad = (m == float(PAD_TOKEN)) | (m == float(MASK_TOKEN))
    keep = jnp.where(bad, 0.0, 1.0).astype(jnp.float32)
    iota = lax.broadcasted_iota(jnp.int32, (BBLK, SEQ, DEPTH), 2)
    out_ref[...] = jnp.where(iota == xi[:, :, None], keep[:, :, None], 0.0)


@jax.jit
def _masked_ohe(x, mask):
    return pl.pallas_call(
        _ohe_tc_body,
        grid=(GRID,),
        in_specs=[
            pl.BlockSpec((BBLK, SEQ), lambda i: (i, 0)),
            pl.BlockSpec((BBLK, SEQ), lambda i: (i, 0)),
        ],
        out_specs=pl.BlockSpec((BBLK, SEQ, DEPTH), lambda i: (i, 0, 0)),
        out_shape=jax.ShapeDtypeStruct((BATCH, SEQ, DEPTH), jnp.float32),
        compiler_params=pltpu.CompilerParams(
            dimension_semantics=("parallel",),
        ),
    )(x, mask)


def kernel(x, mask):
    return _masked_ohe(x.astype(jnp.int32), mask.astype(jnp.float32))


# speedup vs baseline: 1.0049x; 1.0049x over previous
import jax
import jax.numpy as jnp
from jax import lax
from jax.experimental import pallas as pl
from jax.experimental.pallas import tpu as pltpu

VOCAB_SIZE = 1000
DEPTH = VOCAB_SIZE + 1
MASK_TOKEN = -1
PAD_TOKEN = -2

BATCH = 1024
SEQ = 50
BBLK = 16
GRID = BATCH // BBLK
NBUF = 6


def _ohe_tc_body(x_ref, m_ref, out_ref, scratch, sems):
    i = pl.program_id(0)
    slot = lax.rem(i, NBUF)

    @pl.when(i >= NBUF)
    def _():
        prev = i - NBUF
        pltpu.make_async_copy(
            scratch.at[slot],
            out_ref.at[pl.ds(prev * BBLK, BBLK)],
            sems.at[slot],
        ).wait()

    xi = x_ref[...]
    xi = jnp.where(xi == PAD_TOKEN, VOCAB_SIZE, xi)
    m = m_ref[...]
    bad = (m == float(PAD_TOKEN)) | (m == float(MASK_TOKEN))
    keep = jnp.where(bad, 0.0, 1.0).astype(jnp.float32)
    iota = lax.broadcasted_iota(jnp.int32, (BBLK, SEQ, DEPTH), 2)
    scratch[slot] = jnp.where(iota == xi[:, :, None], keep[:, :, None], 0.0)

    pltpu.make_async_copy(
        scratch.at[slot],
        out_ref.at[pl.ds(i * BBLK, BBLK)],
        sems.at[slot],
    ).start()

    @pl.when(i == GRID - 1)
    def _():
        for b in range(NBUF):
            step = GRID - NBUF + b
            s = lax.rem(jnp.int32(step), NBUF)
            pltpu.make_async_copy(
                scratch.at[s],
                out_ref.at[pl.ds(step * BBLK, BBLK)],
                sems.at[s],
            ).wait()


@jax.jit
def _masked_ohe(x, mask):
    return pl.pallas_call(
        _ohe_tc_body,
        grid=(GRID,),
        in_specs=[
            pl.BlockSpec((BBLK, SEQ), lambda i: (i, 0)),
            pl.BlockSpec((BBLK, SEQ), lambda i: (i, 0)),
        ],
        out_specs=pl.BlockSpec(memory_space=pl.ANY),
        out_shape=jax.ShapeDtypeStruct((BATCH, SEQ, DEPTH), jnp.float32),
        scratch_shapes=[
            pltpu.VMEM((NBUF, BBLK, SEQ, DEPTH), jnp.float32),
            pltpu.SemaphoreType.DMA((NBUF,)),
        ],
        compiler_params=pltpu.CompilerParams(
            dimension_semantics=("arbitrary",),
        ),
    )(x, mask)


def kernel(x, mask):
    return _masked_ohe(x.astype(jnp.int32), mask.astype(jnp.float32))


# speedup vs baseline: 4.7354x; 4.7123x over previous
import jax
import jax.numpy as jnp
from jax import lax
from jax.experimental import pallas as pl
from jax.experimental.pallas import tpu as pltpu

VOCAB_SIZE = 1000
DEPTH = VOCAB_SIZE + 1
MASK_TOKEN = -1
PAD_TOKEN = -2

BATCH = 1024
SEQ = 50


def _ohe_tc_body(x_ref, m_ref, out_ref):
    t = pl.program_id(0)
    xi = x_ref[pl.ds(t, 1), :]
    xi = jnp.where(xi == PAD_TOKEN, VOCAB_SIZE, xi)
    m = m_ref[pl.ds(t, 1), :]
    bad = (m == float(PAD_TOKEN)) | (m == float(MASK_TOKEN))
    keep = jnp.where(bad, 0.0, 1.0).astype(jnp.float32)
    iota = lax.broadcasted_iota(jnp.int32, (DEPTH, BATCH), 0)
    out_ref[0] = jnp.where(iota == xi, keep, 0.0)


@jax.jit
def _masked_ohe(x, mask):
    out_t = pl.pallas_call(
        _ohe_tc_body,
        grid=(SEQ,),
        in_specs=[
            pl.BlockSpec((SEQ, BATCH), lambda t: (0, 0)),
            pl.BlockSpec((SEQ, BATCH), lambda t: (0, 0)),
        ],
        out_specs=pl.BlockSpec((1, DEPTH, BATCH), lambda t: (t, 0, 0)),
        out_shape=jax.ShapeDtypeStruct((SEQ, DEPTH, BATCH), jnp.float32),
        compiler_params=pltpu.CompilerParams(
            dimension_semantics=("parallel",),
        ),
    )(x.T, mask.T)
    return jnp.transpose(out_t, (2, 0, 1))


def kernel(x, mask):
    return _masked_ohe(x.astype(jnp.int32), mask.astype(jnp.float32))
